# baseline (speedup 1.0000x reference)
import functools

import jax
import jax.numpy as jnp
from jax.experimental import pallas as pl
from jax.experimental.pallas import tpu as pltpu


def _round_up(v, m):
    return ((v + m - 1) // m) * m


def _pipeline_kernel(x_hbm, w_hbm, b_ref, g_ref, beta_ref, o_hbm,
                     xbuf, obuf, wf32, wbf, in_sem, out_sem, w_sem,
                     *, tm, steps, eps, true_out_dim, in_pad):
    core = pl.program_id(0)
    base = core * steps * tm

    def dma_in(slot, step):
        pltpu.make_async_copy(
            x_hbm.at[pl.ds(base + step * tm, tm), :],
            xbuf.at[slot], in_sem.at[slot]).start()

    def wait_in(slot):
        pltpu.make_async_copy(
            x_hbm.at[pl.ds(0, tm), :],
            xbuf.at[slot], in_sem.at[slot]).wait()

    def dma_out(slot, step):
        pltpu.make_async_copy(
            obuf.at[slot],
            o_hbm.at[pl.ds(base + step * tm, tm), :], out_sem.at[slot]).start()

    def wait_out(slot):
        pltpu.make_async_copy(
            obuf.at[slot],
            o_hbm.at[pl.ds(0, tm), :], out_sem.at[slot]).wait()

    dma_in(0, 0)
    pltpu.make_async_copy(w_hbm, wf32, w_sem).start()
    pltpu.make_async_copy(w_hbm, wf32, w_sem).wait()
    wbf[...] = wf32[...].astype(jnp.bfloat16)

    inv_d = 1.0 / float(true_out_dim)

    def body(step, _):
        cur = jax.lax.rem(step, 2)
        nxt = jax.lax.rem(step + 1, 2)

        @pl.when(step + 1 < steps)
        def _():
            dma_in(nxt, step + 1)

        wait_in(cur)

        @pl.when(step >= 2)
        def _():
            wait_out(cur)

        xb = xbuf[cur].astype(jnp.bfloat16)
        y = jnp.dot(xb, wbf[...], preferred_element_type=jnp.float32)
        y = y + b_ref[...]
        s1 = jnp.sum(y, axis=-1, keepdims=True)
        s2 = jnp.sum(y * y, axis=-1, keepdims=True)
        mean = s1 * inv_d
        var = jnp.maximum(s2 * inv_d - mean * mean, 0.0)
        inv = jax.lax.rsqrt(var + eps)
        out = (y - mean) * inv * g_ref[...] + beta_ref[...]
        obuf[cur] = jnp.maximum(out, 0.0).astype(obuf.dtype)

        dma_out(cur, step)
        return ()

    jax.lax.fori_loop(0, steps, body, (), unroll=False)
    wait_out(jax.lax.rem(steps - 2, 2))
    wait_out(jax.lax.rem(steps - 1, 2))


def kernel(x, w, b, gamma, beta, *, eps=1e-5):
    n, in_dim = x.shape
    out_dim = w.shape[1]

    in_pad = _round_up(in_dim, 128)
    out_pad = _round_up(out_dim, 128)
    tm = 1024
    n_pad = _round_up(n, 2 * tm)
    steps = n_pad // (2 * tm)

    xp = x
    if (n_pad, in_pad) != x.shape:
        xp = jnp.zeros((n_pad, in_pad), x.dtype).at[:n, :in_dim].set(x)
    wp = w
    if (in_pad, out_pad) != w.shape:
        wp = jnp.zeros((in_pad, out_pad), w.dtype).at[:in_dim, :out_dim].set(w)
    bp = b.astype(jnp.float32)
    gp = gamma.astype(jnp.float32)
    betap = beta.astype(jnp.float32)
    if out_pad != out_dim:
        bp = jnp.zeros((1, out_pad), jnp.float32).at[:, :out_dim].set(bp)
        gp = jnp.ones((1, out_pad), jnp.float32).at[:, :out_dim].set(gp)
        betap = jnp.zeros((1, out_pad), jnp.float32).at[:, :out_dim].set(betap)

    body = functools.partial(_pipeline_kernel, tm=tm, steps=steps, eps=eps,
                             true_out_dim=out_dim, in_pad=in_pad)
    y = pl.pallas_call(
        body,
        out_shape=jax.ShapeDtypeStruct((n_pad, out_pad), x.dtype),
        grid=(2,),
        in_specs=[
            pl.BlockSpec(memory_space=pl.ANY),
            pl.BlockSpec(memory_space=pl.ANY),
            pl.BlockSpec((1, out_pad), lambda c: (0, 0)),
            pl.BlockSpec((1, out_pad), lambda c: (0, 0)),
            pl.BlockSpec((1, out_pad), lambda c: (0, 0)),
        ],
        out_specs=pl.BlockSpec(memory_space=pl.ANY),
        scratch_shapes=[
            pltpu.VMEM((2, tm, in_pad), jnp.float32),
            pltpu.VMEM((2, tm, out_pad), jnp.float32),
            pltpu.VMEM((in_pad, out_pad), jnp.float32),
            pltpu.VMEM((in_pad, out_pad), jnp.bfloat16),
            pltpu.SemaphoreType.DMA((2,)),
            pltpu.SemaphoreType.DMA((2,)),
            pltpu.SemaphoreType.DMA,
        ],
        compiler_params=pltpu.CompilerParams(
            dimension_semantics=("parallel",),
            vmem_limit_bytes=64 * 1024 * 1024,
        ),
    )(xp, wp, bp, gp, betap)

    if (n_pad, out_pad) != (n, out_dim):
        y = y[:n, :out_dim]
    return y
